# initial kernel scaffold (unmeasured)
import jax
import jax.numpy as jnp
from jax import lax
from jax.experimental import pallas as pl
from jax.experimental.pallas import tpu as pltpu

N_DEV = 4
M_PER = 1024
K = 4096
N_PER = 2048


def _gelu(y):
    c = 0.7978845608028654
    return 0.5 * y * (1.0 + jnp.tanh(c * (y + 0.044715 * y * y * y)))


def kernel(x, w_mat):
    x = x.astype(jnp.bfloat16)

    def body(x_ref, w_ref, out_ref, w_f32, w_bf16, send_buf,
             load_sem, send_sems, recv_sems):
        my = lax.axis_index("i")

        barrier = pltpu.get_barrier_semaphore()
        for off in (1, 2, 3):
            pl.semaphore_signal(
                barrier, inc=1,
                device_id=((my + off) % N_DEV,),
                device_id_type=pl.DeviceIdType.MESH,
            )
        pl.semaphore_wait(barrier, N_DEV - 1)

        send_rdmas = []
        for s in (1, 3, 2, 0):
            j = (my + s) % N_DEV
            load = pltpu.make_async_copy(
                w_ref.at[:, pl.ds(j * N_PER, N_PER)], w_f32, load_sem)
            load.start()
            load.wait()
            w_bf16[...] = w_f32[...].astype(jnp.bfloat16)
            g = _gelu(jnp.dot(x_ref[...], w_bf16[...],
                              preferred_element_type=jnp.float32))
            g = g.astype(jnp.bfloat16)
            if s == 0:
                out_ref[pl.ds(my * M_PER, M_PER)] = g
            else:
                sb = s - 1
                send_buf[sb] = g
                rdma = pltpu.make_async_remote_copy(
                    src_ref=send_buf.at[sb],
                    dst_ref=out_ref.at[pl.ds(my * M_PER, M_PER)],
                    send_sem=send_sems.at[sb],
                    recv_sem=recv_sems.at[s],
                    device_id=(j,),
                    device_id_type=pl.DeviceIdType.MESH,
                )
                rdma.start()
                send_rdmas.append(rdma)

        for r in send_rdmas:
            r.wait_send()

        for s in (1, 2, 3):
            p = (my - s) % N_DEV
            recv = pltpu.make_async_remote_copy(
                src_ref=send_buf.at[0],
                dst_ref=out_ref.at[pl.ds(p * M_PER, M_PER)],
                send_sem=send_sems.at[0],
                recv_sem=recv_sems.at[s],
                device_id=(p,),
                device_id_type=pl.DeviceIdType.MESH,
            )
            recv.wait_recv()

    return pl.pallas_call(
        body,
        out_shape=jax.ShapeDtypeStruct((N_DEV * M_PER, N_PER), jnp.bfloat16),
        in_specs=[
            pl.BlockSpec(memory_space=pltpu.VMEM),
            pl.BlockSpec(memory_space=pltpu.ANY),
        ],
        out_specs=pl.BlockSpec(memory_space=pltpu.VMEM),
        scratch_shapes=[
            pltpu.VMEM((K, N_PER), jnp.float32),
            pltpu.VMEM((K, N_PER), jnp.bfloat16),
            pltpu.VMEM((3, M_PER, N_PER), jnp.bfloat16),
            pltpu.SemaphoreType.DMA,
            pltpu.SemaphoreType.DMA((3,)),
            pltpu.SemaphoreType.DMA((4,)),
        ],
        compiler_params=pltpu.CompilerParams(
            collective_id=0,
            vmem_limit_bytes=128 * 1024 * 1024,
        ),
    )(x, w_mat)


# baseline (device time: 172668 ns/iter reference)
import jax
import jax.numpy as jnp
from jax import lax
from jax.experimental import pallas as pl
from jax.experimental.pallas import tpu as pltpu

N_DEV = 4
M_PER = 1024
K = 4096
N_PER = 2048
QN = 512
NQ = N_PER // QN
NCH = N_DEV * NQ


def _gelu(y):
    c = 0.7978845608028654
    return 0.5 * y * (1.0 + jnp.tanh(c * (y + 0.044715 * y * y * y)))


def kernel(x, w_mat):
    x = x.astype(jnp.bfloat16)

    def body(x_ref, w_ref, out_ref, w_f32, w_bf16, send_buf,
             load_sems, send_sems, recv_sems):
        my = lax.axis_index("i")

        barrier = pltpu.get_barrier_semaphore()
        for off in (1, 2, 3):
            pl.semaphore_signal(
                barrier, inc=1,
                device_id=((my + off) % N_DEV,),
                device_id_type=pl.DeviceIdType.MESH,
            )
        pl.semaphore_wait(barrier, N_DEV - 1)

        def load_for(idx, buf):
            t = idx // NQ
            q = lax.rem(idx, NQ)
            s = lax.rem(t + 1, N_DEV)
            j = lax.rem(my + s, N_DEV)
            return pltpu.make_async_copy(
                w_ref.at[:, pl.ds(j * N_PER + q * QN, QN)],
                w_f32.at[buf],
                load_sems.at[buf],
            )

        load_for(0, 0).start()

        def step(idx, carry):
            t = idx // NQ
            q = lax.rem(idx, NQ)
            s = lax.rem(t + 1, N_DEV)
            j = lax.rem(my + s, N_DEV)
            buf = lax.rem(idx, 2)

            load_for(idx, buf).wait()

            @pl.when(idx + 1 < NCH)
            def _():
                load_for(idx + 1, 1 - buf).start()

            w_bf16[...] = w_f32[buf].astype(jnp.bfloat16)
            g = _gelu(jnp.dot(x_ref[...], w_bf16[...],
                              preferred_element_type=jnp.float32))
            g = g.astype(jnp.bfloat16)

            @pl.when(s == 0)
            def _():
                out_ref[pl.ds(my * M_PER, M_PER), pl.ds(q * QN, QN)] = g

            @pl.when(s != 0)
            def _():
                send_buf[t, :, pl.ds(q * QN, QN)] = g

                @pl.when(q == NQ - 1)
                def _():
                    rdma = pltpu.make_async_remote_copy(
                        src_ref=send_buf.at[t],
                        dst_ref=out_ref.at[pl.ds(my * M_PER, M_PER)],
                        send_sem=send_sems.at[t],
                        recv_sem=recv_sems.at[s],
                        device_id=(j,),
                        device_id_type=pl.DeviceIdType.MESH,
                    )
                    rdma.start()

            return carry

        lax.fori_loop(0, NCH, step, 0)

        for t in range(N_DEV - 1):
            drain = pltpu.make_async_remote_copy(
                src_ref=send_buf.at[t],
                dst_ref=out_ref.at[pl.ds(0, M_PER)],
                send_sem=send_sems.at[t],
                recv_sem=recv_sems.at[0],
                device_id=(0,),
                device_id_type=pl.DeviceIdType.MESH,
            )
            drain.wait_send()

        for s in (1, 2, 3):
            p = (my - s) % N_DEV
            recv = pltpu.make_async_remote_copy(
                src_ref=send_buf.at[0],
                dst_ref=out_ref.at[pl.ds(p * M_PER, M_PER)],
                send_sem=send_sems.at[0],
                recv_sem=recv_sems.at[s],
                device_id=(p,),
                device_id_type=pl.DeviceIdType.MESH,
            )
            recv.wait_recv()

    return pl.pallas_call(
        body,
        out_shape=jax.ShapeDtypeStruct((N_DEV * M_PER, N_PER), jnp.bfloat16),
        in_specs=[
            pl.BlockSpec(memory_space=pltpu.MemorySpace.VMEM),
            pl.BlockSpec(memory_space=pl.ANY),
        ],
        out_specs=pl.BlockSpec(memory_space=pltpu.MemorySpace.VMEM),
        scratch_shapes=[
            pltpu.VMEM((2, K, QN), jnp.float32),
            pltpu.VMEM((K, QN), jnp.bfloat16),
            pltpu.VMEM((3, M_PER, N_PER), jnp.bfloat16),
            pltpu.SemaphoreType.DMA((2,)),
            pltpu.SemaphoreType.DMA((3,)),
            pltpu.SemaphoreType.DMA((4,)),
        ],
        compiler_params=pltpu.CompilerParams(
            collective_id=0,
            vmem_limit_bytes=62 * 1024 * 1024,
        ),
    )(x, w_mat)


# device time: 132612 ns/iter; 1.3021x vs baseline; 1.3021x over previous
import os

import jax
import jax.numpy as jnp
from jax import lax
from jax.experimental import pallas as pl
from jax.experimental.pallas import tpu as pltpu

_COMM = os.environ.get("KERNEL_NO_COMM") != "1"
_CONVERT = os.environ.get("KERNEL_NO_CONVERT") != "1"
_LOAD = os.environ.get("KERNEL_NO_LOAD") != "1"
_GELU = os.environ.get("KERNEL_NO_GELU") != "1"

N_DEV = 4
M_PER = 1024
K = 4096
N_PER = 2048
QN = int(os.environ.get("KERNEL_QN", "256"))
NQ = N_PER // QN
NCH = N_DEV * NQ


def _gelu(y):
    c = 0.7978845608028654
    return 0.5 * y * (1.0 + jnp.tanh(c * (y + 0.044715 * y * y * y)))


def kernel(x, w_mat):
    x = x.astype(jnp.bfloat16)

    def body(x_ref, w_ref, out_ref, w_f32, w_bf16, send_buf,
             load_sems, send_sems, recv_sems):
        my = lax.axis_index("i")

        NREM = (N_DEV - 1) * NQ

        def chunk_tq(idx):
            t = jnp.where(idx < NREM, lax.rem(idx, N_DEV - 1), N_DEV - 1)
            q = jnp.where(idx < NREM, idx // (N_DEV - 1), idx - NREM)
            return t, q

        def step_offset(t):
            return jnp.where(t == 0, 2, jnp.where(t == 1, 1,
                             jnp.where(t == 2, 3, 0)))

        def load_for(idx, buf):
            t, q = chunk_tq(idx)
            s = step_offset(t)
            j = lax.rem(my + s, N_DEV)
            return pltpu.make_async_copy(
                w_ref.at[:, pl.ds(j * N_PER + q * QN, QN)],
                w_f32.at[buf],
                load_sems.at[buf],
            )

        if _LOAD:
            load_for(0, 0).start()

        barrier = pltpu.get_barrier_semaphore()
        for off in (1, 2, 3):
            pl.semaphore_signal(
                barrier, inc=1,
                device_id=((my + off) % N_DEV,),
                device_id_type=pl.DeviceIdType.MESH,
            )
        pl.semaphore_wait(barrier, N_DEV - 1)

        def step(idx, carry):
            t, q = chunk_tq(idx)
            s = step_offset(t)
            j = lax.rem(my + s, N_DEV)
            buf = lax.rem(idx, 2)

            if _LOAD:
                load_for(idx, buf).wait()

                @pl.when(idx + 1 < NCH)
                def _():
                    load_for(idx + 1, 1 - buf).start()

            if _LOAD and _CONVERT:
                w_bf16[...] = w_f32[buf].astype(jnp.bfloat16)
            y = jnp.dot(x_ref[...], w_bf16[...],
                        preferred_element_type=jnp.float32)
            g = (_gelu(y) if _GELU else y).astype(jnp.bfloat16)

            @pl.when(s == 0)
            def _():
                out_ref[pl.ds(my * M_PER, M_PER), pl.ds(q * QN, QN)] = g

            @pl.when(s != 0)
            def _():
                send_buf[t, :, pl.ds(q * QN, QN)] = g

                @pl.when(_COMM)
                def _():
                    rdma = pltpu.make_async_remote_copy(
                        src_ref=send_buf.at[t, :, pl.ds(q * QN, QN)],
                        dst_ref=out_ref.at[pl.ds(my * M_PER, M_PER),
                                           pl.ds(q * QN, QN)],
                        send_sem=send_sems.at[idx],
                        recv_sem=recv_sems.at[s * NQ + q],
                        device_id=(j,),
                        device_id_type=pl.DeviceIdType.MESH,
                    )
                    rdma.start()

            return carry

        lax.fori_loop(0, NCH, step, 0)

        for tq in range(NREM if _COMM else 0):
            ts, qs = tq % (N_DEV - 1), tq // (N_DEV - 1)
            drain = pltpu.make_async_remote_copy(
                src_ref=send_buf.at[ts, :, pl.ds(qs * QN, QN)],
                dst_ref=out_ref.at[pl.ds(0, M_PER),
                                   pl.ds(qs * QN, QN)],
                send_sem=send_sems.at[tq],
                recv_sem=recv_sems.at[0],
                device_id=(0,),
                device_id_type=pl.DeviceIdType.MESH,
            )
            drain.wait_send()

        for qs in range(NQ if _COMM else 0):
            for s in (2, 1, 3):
                p = (my - s) % N_DEV
                recv = pltpu.make_async_remote_copy(
                    src_ref=send_buf.at[0, :, pl.ds(qs * QN, QN)],
                    dst_ref=out_ref.at[pl.ds(p * M_PER, M_PER),
                                       pl.ds(qs * QN, QN)],
                    send_sem=send_sems.at[0],
                    recv_sem=recv_sems.at[s * NQ + qs],
                    device_id=(p,),
                    device_id_type=pl.DeviceIdType.MESH,
                )
                recv.wait_recv()

    return pl.pallas_call(
        body,
        out_shape=jax.ShapeDtypeStruct((N_DEV * M_PER, N_PER), jnp.bfloat16),
        in_specs=[
            pl.BlockSpec(memory_space=pltpu.MemorySpace.VMEM),
            pl.BlockSpec(memory_space=pl.ANY),
        ],
        out_specs=pl.BlockSpec(memory_space=pltpu.MemorySpace.VMEM),
        scratch_shapes=[
            pltpu.VMEM((2, K, QN) if _LOAD else (2, 8, 128), jnp.float32),
            pltpu.VMEM((K, QN), jnp.bfloat16),
            pltpu.VMEM((3, M_PER, N_PER), jnp.bfloat16),
            pltpu.SemaphoreType.DMA((2,)),
            pltpu.SemaphoreType.DMA((NCH,)),
            pltpu.SemaphoreType.DMA((N_DEV * NQ,)),
        ],
        compiler_params=pltpu.CompilerParams(
            collective_id=0,
            vmem_limit_bytes=62 * 1024 * 1024,
        ),
    )(x, w_mat)
